# Initial kernel scaffold; baseline (speedup 1.0000x reference)
import jax
import jax.numpy as jnp
from jax.experimental import pallas as pl


def kernel(x, edge_index, edge_weight, W_enc, W_dec, gamma, beta):
    raise NotImplementedError("write your pallas kernel here")


# speedup vs baseline: 4.0272x; 4.0272x over previous
import functools

import jax
import jax.numpy as jnp
from jax import lax
from jax.experimental import pallas as pl
from jax.experimental.pallas import tpu as pltpu
from jax.experimental.pallas import tpu_sc as plsc

N = 10000
E = 320000
D = 128
ALPHA = 0.1
K_STEPS = 10
N_LAYERS = 2

NC = 2
NS = 16
NW = NC * NS
EPT = E // NW
CHK = 80
NCHUNK = EPT // CHK
NPAD = 10240
ROWS_PT = NPAD // NS
OB = 128


def _sc_round_body(x_hbm, src_hbm, dst_hbm, w_hbm, out_hbm,
                   src_v, dst_v, w_v, rows_v, ob_v, acc_sh, sem):
    c = lax.axis_index("c")
    s = lax.axis_index("s")
    wid = c * NS + s
    z16 = jnp.zeros((16,), jnp.float32)

    def zb(i, carry):
        for k in range(D // 16):
            ob_v[i, pl.ds(k * 16, 16)] = z16
        return carry
    lax.fori_loop(0, OB, zb, 0)
    r0 = s * ROWS_PT
    for k in range(ROWS_PT // OB):
        pltpu.sync_copy(ob_v, acc_sh.at[pl.ds(r0 + k * OB, OB)])
    plsc.subcore_barrier()

    e_base = wid * EPT

    def chunk(i, carry):
        e0 = e_base + i * CHK
        pltpu.sync_copy(src_hbm.at[pl.ds(e0, CHK)], src_v)
        pltpu.sync_copy(dst_hbm.at[pl.ds(e0, CHK)], dst_v)
        pltpu.sync_copy(w_hbm.at[pl.ds(e0, CHK)], w_v)
        pltpu.async_copy(x_hbm.at[src_v], rows_v, sem).wait()

        def scale(g, c2):
            wvec = w_v[pl.ds(g * 16, 16)]
            for t in range(16):
                wj = wvec[t]
                j = g * 16 + t
                for k in range(D // 16):
                    sl = pl.ds(k * 16, 16)
                    rows_v[j, sl] = rows_v[j, sl] * wj
            return c2
        lax.fori_loop(0, CHK // 16, scale, 0)

        pltpu.sync_copy(rows_v, acc_sh.at[dst_v], add=True)
        return carry
    lax.fori_loop(0, NCHUNK, chunk, 0)

    plsc.subcore_barrier()
    for k in range(ROWS_PT // OB):
        rr = r0 + k * OB
        pltpu.sync_copy(acc_sh.at[pl.ds(rr, OB)], ob_v)
        pltpu.sync_copy(ob_v, out_hbm.at[c, pl.ds(rr, OB)])


_appnp_round = pl.kernel(
    _sc_round_body,
    out_type=jax.ShapeDtypeStruct((NC, NPAD, D), jnp.float32),
    mesh=plsc.VectorSubcoreMesh(core_axis_name="c", subcore_axis_name="s"),
    scratch_types=[
        pltpu.VMEM((CHK,), jnp.int32),
        pltpu.VMEM((CHK,), jnp.int32),
        pltpu.VMEM((CHK,), jnp.float32),
        pltpu.VMEM((CHK, D), jnp.float32),
        pltpu.VMEM((OB, D), jnp.float32),
        pltpu.VMEM_SHARED((NPAD, D), jnp.float32),
        pltpu.SemaphoreType.DMA,
    ],
)


_BLK = 1000


def _gelu(h):
    return 0.5 * h * (1.0 + lax.erf(h * (2.0 ** -0.5)))


def _ln(h, g, b):
    m = jnp.mean(h, axis=-1, keepdims=True)
    v = jnp.mean((h - m) * (h - m), axis=-1, keepdims=True)
    return (h - m) * lax.rsqrt(v + 1e-5) * g + b


def _enc_body(x_ref, w_ref, g_ref, b_ref, o_ref):
    h = jnp.dot(x_ref[...], w_ref[...], preferred_element_type=jnp.float32)
    o_ref[...] = _gelu(_ln(h, g_ref[...], b_ref[...]))


def _comb_body(p_ref, h_ref, o_ref):
    o_ref[...] = (1.0 - ALPHA) * (p_ref[0] + p_ref[1]) + ALPHA * h_ref[...]


def _comb_gln_body(p_ref, h_ref, g_ref, b_ref, o_ref):
    xb = (1.0 - ALPHA) * (p_ref[0] + p_ref[1]) + ALPHA * h_ref[...]
    o_ref[...] = _ln(_gelu(xb), g_ref[...], b_ref[...])


def _dec_body(x_ref, w_ref, o_ref):
    o_ref[...] = jnp.dot(x_ref[...], w_ref[...],
                         preferred_element_type=jnp.float32)


_row_spec = pl.BlockSpec((_BLK, D), lambda i: (i, 0))
_mat_spec = pl.BlockSpec((D, D), lambda i: (0, 0))
_vec_spec = pl.BlockSpec((1, D), lambda i: (0, 0))
_par_spec = pl.BlockSpec((NC, _BLK, D), lambda i: (0, i, 0))
_out_shape = jax.ShapeDtypeStruct((N, D), jnp.float32)
_grid = (N // _BLK,)

_enc = pl.pallas_call(
    _enc_body, grid=_grid,
    in_specs=[_row_spec, _mat_spec, _vec_spec, _vec_spec],
    out_specs=_row_spec, out_shape=_out_shape)

_comb = pl.pallas_call(
    _comb_body, grid=_grid,
    in_specs=[_par_spec, _row_spec],
    out_specs=_row_spec, out_shape=_out_shape)

_comb_gln = pl.pallas_call(
    _comb_gln_body, grid=_grid,
    in_specs=[_par_spec, _row_spec, _vec_spec, _vec_spec],
    out_specs=_row_spec, out_shape=_out_shape)

_dec = pl.pallas_call(
    _dec_body, grid=_grid,
    in_specs=[_row_spec, _mat_spec],
    out_specs=_row_spec, out_shape=_out_shape)


def kernel(x, edge_index, edge_weight, W_enc, W_dec, gamma, beta):
    src = edge_index[0]
    dst = edge_index[1]
    g2 = gamma.reshape(1, D)
    b2 = beta.reshape(1, D)
    h = _enc(x, W_enc.T, g2, b2)
    for _ in range(N_LAYERS):
        xc = h
        for k in range(K_STEPS):
            parts = _appnp_round(xc, src, dst, edge_weight)
            if k < K_STEPS - 1:
                xc = _comb(parts, h)
            else:
                xc = _comb_gln(parts, h, g2, b2)
        h = xc
    return _dec(h, W_dec.T)
